# baseline (speedup 1.0000x reference)
import functools
import jax
import jax.numpy as jnp
from jax import lax
from jax.experimental import pallas as pl
from jax.experimental.pallas import tpu as pltpu
from jax.experimental.pallas import tpu_sc as plsc


@functools.cache
def _make_gather(V, D, B):
    info = plsc.get_sparse_core_info()
    NC, NS = info.num_cores, info.num_subcores
    NW = NC * NS
    assert B % (8 * NW) == 0
    b_per_w = B // NW
    CHUNK = 128
    n_chunks = max(1, -(-b_per_w // CHUNK))
    chunk = b_per_w // n_chunks
    assert chunk * n_chunks == b_per_w and chunk <= 128
    mesh = plsc.VectorSubcoreMesh(core_axis_name="c", subcore_axis_name="s")

    @functools.partial(
        pl.kernel,
        mesh=mesh,
        out_type=jax.ShapeDtypeStruct((B, D), jnp.float32),
        scratch_types=(
            [pltpu.VMEM((n_chunks, chunk), jnp.int32),
             pltpu.VMEM((b_per_w, D), jnp.float32)]
            + [pltpu.SemaphoreType.DMA] * n_chunks
            + [pltpu.SemaphoreType.DMA]
        ),
    )
    def k(table_hbm, idx_hbm, out_hbm, idx_v, rows_v, *sems):
        gsems, osem = sems[:n_chunks], sems[n_chunks]
        wid = lax.axis_index("s") * NC + lax.axis_index("c")
        base = wid * b_per_w
        pltpu.sync_copy(idx_hbm.at[wid], idx_v)
        gathers = []
        for j in range(n_chunks):
            gathers.append(pltpu.async_copy(
                table_hbm.at[idx_v.at[j]],
                rows_v.at[pl.ds(j * chunk, chunk)],
                gsems[j]))
        outs = []
        for j in range(n_chunks):
            gathers[j].wait()
            outs.append(pltpu.async_copy(
                rows_v.at[pl.ds(j * chunk, chunk)],
                out_hbm.at[pl.ds(base + j * chunk, chunk)],
                osem))
        for c in outs:
            c.wait()

    return k, NW, n_chunks, chunk


def kernel(labels, table):
    B, = labels.shape
    V, D = table.shape
    k, NW, n_chunks, chunk = _make_gather(V, D, B)
    idx = labels.astype(jnp.int32).reshape(NW, n_chunks, chunk)
    return k(table, idx)


# speedup vs baseline: 1.0118x; 1.0118x over previous
import functools
import jax
import jax.numpy as jnp
from jax import lax
from jax.experimental import pallas as pl
from jax.experimental.pallas import tpu as pltpu
from jax.experimental.pallas import tpu_sc as plsc


@functools.cache
def _make_gather(V, D, B):
    info = plsc.get_sparse_core_info()
    NC, NS = info.num_cores, info.num_subcores
    NW = NC * NS
    assert B % (8 * NW) == 0
    b_per_w = B // NW
    CHUNK = 512
    n_chunks = max(1, -(-b_per_w // CHUNK))
    chunk = b_per_w // n_chunks
    assert chunk * n_chunks == b_per_w
    mesh = plsc.VectorSubcoreMesh(core_axis_name="c", subcore_axis_name="s")

    @functools.partial(
        pl.kernel,
        mesh=mesh,
        out_type=jax.ShapeDtypeStruct((B, D), jnp.float32),
        scratch_types=(
            [pltpu.VMEM((n_chunks, chunk), jnp.int32),
             pltpu.VMEM((b_per_w, D), jnp.float32)]
            + [pltpu.SemaphoreType.DMA] * n_chunks
            + [pltpu.SemaphoreType.DMA]
        ),
    )
    def k(table_hbm, idx_hbm, out_hbm, idx_v, rows_v, *sems):
        gsems, osem = sems[:n_chunks], sems[n_chunks]
        wid = lax.axis_index("s") * NC + lax.axis_index("c")
        base = wid * b_per_w
        pltpu.sync_copy(idx_hbm.at[wid], idx_v)
        gathers = []
        for j in range(n_chunks):
            gathers.append(pltpu.async_copy(
                table_hbm.at[idx_v.at[j]],
                rows_v.at[pl.ds(j * chunk, chunk)],
                gsems[j]))
        outs = []
        for j in range(n_chunks):
            gathers[j].wait()
            outs.append(pltpu.async_copy(
                rows_v.at[pl.ds(j * chunk, chunk)],
                out_hbm.at[pl.ds(base + j * chunk, chunk)],
                osem))
        for c in outs:
            c.wait()

    return k, NW, n_chunks, chunk


def kernel(labels, table):
    B, = labels.shape
    V, D = table.shape
    k, NW, n_chunks, chunk = _make_gather(V, D, B)
    idx = labels.astype(jnp.int32).reshape(NW, n_chunks, chunk)
    return k(table, idx)


# speedup vs baseline: 1.0137x; 1.0019x over previous
import functools
import jax
import jax.numpy as jnp
from jax import lax
from jax.experimental import pallas as pl
from jax.experimental.pallas import tpu as pltpu
from jax.experimental.pallas import tpu_sc as plsc


@functools.cache
def _make_gather(V, D, B):
    info = plsc.get_sparse_core_info()
    NC, NS = info.num_cores, info.num_subcores
    NW = NC * NS
    assert B % (8 * NW) == 0
    b_per_w = B // NW
    mesh = plsc.VectorSubcoreMesh(core_axis_name="c", subcore_axis_name="s")

    @functools.partial(
        pl.kernel,
        mesh=mesh,
        out_type=jax.ShapeDtypeStruct((B, D), jnp.float32),
        scratch_types=[
            pltpu.VMEM((b_per_w,), jnp.int32),
            pltpu.VMEM((b_per_w, D), jnp.float32),
            pltpu.SemaphoreType.DMA,
        ],
    )
    def k(table_hbm, idx_hbm, out_hbm, idx_v, rows_v, sem):
        wid = lax.axis_index("s") * NC + lax.axis_index("c")
        base = wid * b_per_w
        pltpu.sync_copy(idx_hbm.at[pl.ds(base, b_per_w)], idx_v)
        pltpu.async_copy(table_hbm.at[idx_v], rows_v, sem).wait()
        pltpu.sync_copy(rows_v, out_hbm.at[pl.ds(base, b_per_w)])

    return k


def kernel(labels, table):
    B, = labels.shape
    V, D = table.shape
    k = _make_gather(V, D, B)
    return k(table, labels.astype(jnp.int32))
